# baseline (speedup 1.0000x reference)
import functools

import jax
import jax.numpy as jnp
import numpy as np
from jax import lax
from jax.experimental import pallas as pl
from jax.experimental.pallas import tpu as pltpu
from jax.experimental.pallas import tpu_sc as plsc

C = 256
K = 16
N = 2048
SCALE = float(np.sqrt(K))

SC_CORES = 2
SC_SUBCORES = 16
NW = SC_CORES * SC_SUBCORES
LANES = 16

G = 4


def _proj_body(f_ref, xyz_ref, wft_ref, bft_ref, wco_ref, bco_ref,
               wfe_ref, bfe_ref, t_ref, pft_ref, qq_ref):
    f = f_ref[...]
    xyz = xyz_ref[...]
    hp = lax.Precision.HIGHEST
    pft = jnp.dot(f, wft_ref[...], precision=hp,
                  preferred_element_type=jnp.float32) + bft_ref[...]
    pfe = jnp.dot(f, wfe_ref[...], precision=hp,
                  preferred_element_type=jnp.float32)
    pco = jnp.dot(xyz, wco_ref[...], precision=hp,
                  preferred_element_type=jnp.float32)
    inv_s = jnp.float32(1.0 / SCALE)
    t_ref[:, 0:C] = pco * inv_s
    t_ref[:, C:2 * C] = pfe
    pft_ref[...] = pft
    qq_ref[:, 0:C] = (pco - bco_ref[...]) * inv_s
    qq_ref[:, C:2 * C] = pfe - bfe_ref[...]


def _projections(features, xyzp, W_ft, b_ft, W_cop, b_co, W_fe, b_fe):
    total = features.shape[0]
    rb = 512
    grid = (total // rb,)
    row = lambda i: (i, 0)
    rep = lambda i: (0, 0)
    return pl.pallas_call(
        _proj_body,
        grid=grid,
        in_specs=[
            pl.BlockSpec((rb, C), row),
            pl.BlockSpec((rb, 8), row),
            pl.BlockSpec((C, C), rep),
            pl.BlockSpec((1, C), rep),
            pl.BlockSpec((8, C), rep),
            pl.BlockSpec((1, C), rep),
            pl.BlockSpec((C, C), rep),
            pl.BlockSpec((1, C), rep),
        ],
        out_specs=[pl.BlockSpec((rb, 2 * C), row),
                   pl.BlockSpec((rb, C), row),
                   pl.BlockSpec((rb, 2 * C), row)],
        out_shape=[jax.ShapeDtypeStruct((total, 2 * C), jnp.float32),
                   jax.ShapeDtypeStruct((total, C), jnp.float32),
                   jax.ShapeDtypeStruct((total, 2 * C), jnp.float32)],
    )(features, xyzp, W_ft, b_ft, W_cop, b_co, W_fe, b_fe)


def _knn_body(off, rows_ref, pts_ref, x2c_ref, x2r_ref, idx_ref):
    rows = rows_ref[0]
    pts = pts_ref[0]
    x2c = x2c_ref[...]
    x2r = x2r_ref[0]
    dot = lax.dot_general(rows, pts, (((1,), (1,)), ((), ())),
                          precision=lax.Precision.DEFAULT,
                          preferred_element_type=jnp.float32)
    d = x2c + x2r - 2.0 * dot
    rb = d.shape[0]
    iota = lax.broadcasted_iota(jnp.int32, (rb, N), 1)
    big = jnp.int32(N * 4)
    offv = jnp.int32(off)
    for t in range(K):
        m = jnp.min(d, axis=1, keepdims=True)
        am = jnp.min(jnp.where(d == m, iota, big), axis=1, keepdims=True)
        idx_ref[:, t:t + 1] = am + offv
        d = jnp.where(iota == am, jnp.float32(np.inf), d)


def _knn_batch(xyzp_b, x2col_b, x2row_b, off):
    rb = 256
    xyzp3 = xyzp_b.reshape(1, N, 8)
    return pl.pallas_call(
        functools.partial(_knn_body, off),
        grid=(N // rb,),
        in_specs=[
            pl.BlockSpec((1, rb, 8), lambda r: (0, r, 0)),
            pl.BlockSpec((1, N, 8), lambda r: (0, 0, 0)),
            pl.BlockSpec((rb, 1), lambda r: (r, 0)),
            pl.BlockSpec((1, 1, N), lambda r: (0, 0, 0)),
        ],
        out_specs=pl.BlockSpec((rb, K), lambda r: (r, 0)),
        out_shape=jax.ShapeDtypeStruct((N, K), jnp.int32),
    )(xyzp3, xyzp3, x2col_b, x2row_b)


NSET = 4


def _sc_s_point(cofe, qq_v, s_v):
    for cg in range(C // LANES):
        co_sl = pl.ds(cg * LANES, LANES)
        fe_sl = pl.ds(C + cg * LANES, LANES)
        qco = qq_v[0, co_sl]
        qfe = qq_v[0, fe_sl]
        for j in range(K):
            s_v[j, co_sl] = ((cofe[j, co_sl] - qco) *
                             (cofe[j, fe_sl] - qfe))


def _sc_s_body(row_base, t_hbm, pft_hbm, qq_hbm, idxf_hbm, s_hbm, ft_hbm,
               idxf_v, cofe, ftb, qqb, sv, sems, osems):
    wid = lax.axis_index("s") * SC_CORES + lax.axis_index("c")
    npw = s_hbm.shape[0] // K // NW
    base = wid * npw

    pltpu.sync_copy(idxf_hbm.at[pl.ds(base * K, npw * K)], idxf_v)

    def fire(blk, t):
        isl = idxf_v.at[pl.ds(blk * K, K)]
        pltpu.async_copy(t_hbm.at[isl], cofe[t], sems[t])
        pltpu.async_copy(pft_hbm.at[isl], ftb[t], sems[t])
        pltpu.async_copy(qq_hbm.at[pl.ds(row_base + base + blk, 1)],
                         qqb[t], sems[t])

    def wait_in(t):
        pltpu.make_async_copy(t_hbm.at[idxf_v.at[pl.ds(0, K)]],
                              cofe[t], sems[t]).wait()
        pltpu.make_async_copy(pft_hbm.at[idxf_v.at[pl.ds(0, K)]],
                              ftb[t], sems[t]).wait()
        pltpu.make_async_copy(qq_hbm.at[pl.ds(row_base + base, 1)],
                              qqb[t], sems[t]).wait()

    def fire_out(blk, t):
        orow = (base + blk) * K
        pltpu.async_copy(sv[t], s_hbm.at[pl.ds(orow, K)], osems[t])
        pltpu.async_copy(ftb[t], ft_hbm.at[pl.ds(orow, K)], osems[t])

    def wait_out(t):
        pltpu.make_async_copy(sv[t], s_hbm.at[pl.ds(0, K)], osems[t]).wait()
        pltpu.make_async_copy(ftb[t], ft_hbm.at[pl.ds(0, K)], osems[t]).wait()

    fire(0, 0)
    fire(1, 1)

    def si_body(si):
        for t in range(NSET):
            b = si * NSET + t
            tn = (t + 2) % NSET
            @pl.when(b + 2 >= NSET)
            def _():
                wait_out(tn)

            @pl.when(b + 2 < npw)
            def _():
                fire(b + 2, tn)

            wait_in(t)
            _sc_s_point(cofe[t], qqb[t], sv[t])
            fire_out(b, t)

    pl.loop(0, npw // NSET)(si_body)
    wait_out((npw - 2) % NSET)
    wait_out((npw - 1) % NSET)


def _sc_s_batch(t_tab, pft_tab, qq_tab, idx_flat_b, row_base):
    npw = N // NW
    mesh = plsc.VectorSubcoreMesh(core_axis_name="c", subcore_axis_name="s",
                                  num_cores=SC_CORES,
                                  num_subcores=SC_SUBCORES)
    fn = pl.kernel(
        functools.partial(_sc_s_body, row_base),
        out_type=[jax.ShapeDtypeStruct((N * K, C), jnp.float32),
                  jax.ShapeDtypeStruct((N * K, C), jnp.float32)],
        mesh=mesh,
        scratch_types=[
            pltpu.VMEM((npw * K,), jnp.int32),
            [pltpu.VMEM((K, 2 * C), jnp.float32)] * NSET,
            [pltpu.VMEM((K, C), jnp.float32)] * NSET,
            [pltpu.VMEM((1, 2 * C), jnp.float32)] * NSET,
            [pltpu.VMEM((K, C), jnp.float32)] * NSET,
            [pltpu.SemaphoreType.DMA] * NSET,
            [pltpu.SemaphoreType.DMA] * NSET,
        ],
    )
    return fn(t_tab, pft_tab, qq_tab, idx_flat_b)


def _soft_body(s_ref, ft_ref, out_ref):
    x = s_ref[...]
    m = jnp.max(x, axis=1, keepdims=True)
    e = jnp.exp(x - m)
    den = jnp.sum(e, axis=1)
    num = jnp.sum(e * ft_ref[...], axis=1)
    out_ref[...] = num / den


def _softmax_batch(s_b, ft_b):
    pt = 32
    s3 = s_b.reshape(N, K, C)
    ft3 = ft_b.reshape(N, K, C)
    return pl.pallas_call(
        _soft_body,
        grid=(N // pt,),
        in_specs=[pl.BlockSpec((pt, K, C), lambda r: (r, 0, 0)),
                  pl.BlockSpec((pt, K, C), lambda r: (r, 0, 0))],
        out_specs=pl.BlockSpec((pt, C), lambda r: (r, 0)),
        out_shape=jax.ShapeDtypeStruct((N, C), jnp.float32),
    )(s3, ft3)


def _ln_body(attn_ref, f_ref, g_ref, b_ref, out_ref):
    x = attn_ref[...] + f_ref[...]
    mu = jnp.mean(x, axis=-1, keepdims=True)
    var = jnp.mean((x - mu) ** 2, axis=-1, keepdims=True)
    out_ref[...] = (x - mu) / jnp.sqrt(var + 1e-5) * g_ref[...] + b_ref[...]


def _layernorm(attn, features, gamma, beta):
    total = features.shape[0]
    rb = 512
    row = lambda i: (i, 0)
    rep = lambda i: (0, 0)
    return pl.pallas_call(
        _ln_body,
        grid=(total // rb,),
        in_specs=[
            pl.BlockSpec((rb, C), row),
            pl.BlockSpec((rb, C), row),
            pl.BlockSpec((1, C), rep),
            pl.BlockSpec((1, C), rep),
        ],
        out_specs=pl.BlockSpec((rb, C), row),
        out_shape=jax.ShapeDtypeStruct((total, C), jnp.float32),
    )(attn, features, gamma, beta)


def kernel(features, coords, W_ft, b_ft, W_co, b_co, W_fe, b_fe, gamma, beta):
    total = features.shape[0]
    nb = total // N
    xyz = coords[:, 1:4]
    x2 = jnp.sum(xyz * xyz, axis=1)
    xyzp = jnp.pad(xyz, ((0, 0), (0, 5)))
    W_cop = jnp.pad(W_co, ((0, 5), (0, 0)))

    t_tab, pft_tab, qq_tab = _projections(
        features, xyzp, W_ft, b_ft.reshape(1, C), W_cop, b_co.reshape(1, C),
        W_fe, b_fe.reshape(1, C))

    x2col = x2.reshape(total, 1)
    x2row = x2.reshape(nb, 1, 1, N)
    idxs = [_knn_batch(xyzp[b * N:(b + 1) * N],
                       x2col[b * N:(b + 1) * N],
                       x2row[b], b * N) for b in range(nb)]
    attns = []
    for b in range(nb):
        s_b, ft_b = _sc_s_batch(t_tab, pft_tab, qq_tab,
                                idxs[b].reshape(-1), b * N)
        attns.append(_softmax_batch(s_b, ft_b))
    attn = jnp.concatenate(attns, axis=0)

    return _layernorm(attn, features, gamma.reshape(1, C),
                      beta.reshape(1, C))


# speedup vs baseline: 1.0037x; 1.0037x over previous
import functools

import jax
import jax.numpy as jnp
import numpy as np
from jax import lax
from jax.experimental import pallas as pl
from jax.experimental.pallas import tpu as pltpu
from jax.experimental.pallas import tpu_sc as plsc

C = 256
K = 16
N = 2048
SCALE = float(np.sqrt(K))

SC_CORES = 2
SC_SUBCORES = 16
NW = SC_CORES * SC_SUBCORES
LANES = 16

G = 4


def _proj_body(f_ref, xyz_ref, wft_ref, bft_ref, wco_ref, bco_ref,
               wfe_ref, bfe_ref, t_ref, pft_ref, qq_ref):
    f = f_ref[...]
    xyz = xyz_ref[...]
    hp = lax.Precision.HIGHEST
    pft = jnp.dot(f, wft_ref[...], precision=hp,
                  preferred_element_type=jnp.float32) + bft_ref[...]
    pfe = jnp.dot(f, wfe_ref[...], precision=hp,
                  preferred_element_type=jnp.float32)
    pco = jnp.dot(xyz, wco_ref[...], precision=hp,
                  preferred_element_type=jnp.float32)
    inv_s = jnp.float32(1.0 / SCALE)
    t_ref[:, 0:C] = pco * inv_s
    t_ref[:, C:2 * C] = pfe
    pft_ref[...] = pft
    qq_ref[:, 0:C] = (pco - bco_ref[...]) * inv_s
    qq_ref[:, C:2 * C] = pfe - bfe_ref[...]


def _projections(features, xyzp, W_ft, b_ft, W_cop, b_co, W_fe, b_fe):
    total = features.shape[0]
    rb = 512
    grid = (total // rb,)
    row = lambda i: (i, 0)
    rep = lambda i: (0, 0)
    return pl.pallas_call(
        _proj_body,
        grid=grid,
        in_specs=[
            pl.BlockSpec((rb, C), row),
            pl.BlockSpec((rb, 8), row),
            pl.BlockSpec((C, C), rep),
            pl.BlockSpec((1, C), rep),
            pl.BlockSpec((8, C), rep),
            pl.BlockSpec((1, C), rep),
            pl.BlockSpec((C, C), rep),
            pl.BlockSpec((1, C), rep),
        ],
        out_specs=[pl.BlockSpec((rb, 2 * C), row),
                   pl.BlockSpec((rb, C), row),
                   pl.BlockSpec((rb, 2 * C), row)],
        out_shape=[jax.ShapeDtypeStruct((total, 2 * C), jnp.float32),
                   jax.ShapeDtypeStruct((total, C), jnp.float32),
                   jax.ShapeDtypeStruct((total, 2 * C), jnp.float32)],
    )(features, xyzp, W_ft, b_ft, W_cop, b_co, W_fe, b_fe)


def _knn_body(off, rows_ref, pts_ref, x2c_ref, x2r_ref, idx_ref):
    rows = rows_ref[0]
    pts = pts_ref[0]
    x2c = x2c_ref[...]
    x2r = x2r_ref[0]
    dot = lax.dot_general(rows, pts, (((1,), (1,)), ((), ())),
                          precision=lax.Precision.DEFAULT,
                          preferred_element_type=jnp.float32)
    d = x2c + x2r - 2.0 * dot
    rb = d.shape[0]
    iota = lax.broadcasted_iota(jnp.int32, (rb, N), 1)
    big = jnp.int32(N * 4)
    offv = jnp.int32(off)
    for t in range(K):
        m = jnp.min(d, axis=1, keepdims=True)
        am = jnp.min(jnp.where(d == m, iota, big), axis=1, keepdims=True)
        idx_ref[:, t:t + 1] = am + offv
        d = jnp.where(iota == am, jnp.float32(np.inf), d)


def _knn_batch(xyzp_b, x2col_b, x2row_b, off):
    rb = 256
    xyzp3 = xyzp_b.reshape(1, N, 8)
    return pl.pallas_call(
        functools.partial(_knn_body, off),
        grid=(N // rb,),
        in_specs=[
            pl.BlockSpec((1, rb, 8), lambda r: (0, r, 0)),
            pl.BlockSpec((1, N, 8), lambda r: (0, 0, 0)),
            pl.BlockSpec((rb, 1), lambda r: (r, 0)),
            pl.BlockSpec((1, 1, N), lambda r: (0, 0, 0)),
        ],
        out_specs=pl.BlockSpec((rb, K), lambda r: (r, 0)),
        out_shape=jax.ShapeDtypeStruct((N, K), jnp.int32),
    )(xyzp3, xyzp3, x2col_b, x2row_b)


NSET = 4


def _sc_s_point(cofe, qq_v, s_v):
    for cg in range(C // LANES):
        co_sl = pl.ds(cg * LANES, LANES)
        fe_sl = pl.ds(C + cg * LANES, LANES)
        qco = qq_v[0, co_sl]
        qfe = qq_v[0, fe_sl]
        for j in range(K):
            s_v[j, co_sl] = ((cofe[j, co_sl] - qco) *
                             (cofe[j, fe_sl] - qfe))


def _sc_s_body(row_base, t_hbm, pft_hbm, qq_hbm, idxf_hbm, s_hbm, ft_hbm,
               idxf_v, cofe, ftb, qqb, sv, sems, osems):
    wid = lax.axis_index("s") * SC_CORES + lax.axis_index("c")
    npw = s_hbm.shape[0] // K // NW
    base = wid * npw

    pltpu.sync_copy(idxf_hbm.at[pl.ds(base * K, npw * K)], idxf_v)

    def fire(blk, t):
        isl = idxf_v.at[pl.ds(blk * K, K)]
        pltpu.async_copy(t_hbm.at[isl], cofe[t], sems[t])
        pltpu.async_copy(pft_hbm.at[isl], ftb[t], sems[t])
        pltpu.async_copy(qq_hbm.at[pl.ds(row_base + base + blk, 1)],
                         qqb[t], sems[t])

    def wait_in(t):
        pltpu.make_async_copy(t_hbm.at[idxf_v.at[pl.ds(0, K)]],
                              cofe[t], sems[t]).wait()
        pltpu.make_async_copy(pft_hbm.at[idxf_v.at[pl.ds(0, K)]],
                              ftb[t], sems[t]).wait()
        pltpu.make_async_copy(qq_hbm.at[pl.ds(row_base + base, 1)],
                              qqb[t], sems[t]).wait()

    def fire_out(blk, t):
        orow = (base + blk) * K
        pltpu.async_copy(sv[t], s_hbm.at[pl.ds(orow, K)], osems[t])
        pltpu.async_copy(ftb[t], ft_hbm.at[pl.ds(orow, K)], osems[t])

    def wait_out(t):
        pltpu.make_async_copy(sv[t], s_hbm.at[pl.ds(0, K)], osems[t]).wait()
        pltpu.make_async_copy(ftb[t], ft_hbm.at[pl.ds(0, K)], osems[t]).wait()

    fire(0, 0)
    fire(1, 1)

    def si_body(si):
        for t in range(NSET):
            b = si * NSET + t
            tn = (t + 2) % NSET
            @pl.when(b + 2 >= NSET)
            def _():
                wait_out(tn)

            @pl.when(b + 2 < npw)
            def _():
                fire(b + 2, tn)

            wait_in(t)
            _sc_s_point(cofe[t], qqb[t], sv[t])
            fire_out(b, t)

    pl.loop(0, npw // NSET)(si_body)
    wait_out((npw - 2) % NSET)
    wait_out((npw - 1) % NSET)


def _sc_s_batch(t_tab, pft_tab, qq_tab, idx_flat_b, row_base):
    npw = N // NW
    mesh = plsc.VectorSubcoreMesh(core_axis_name="c", subcore_axis_name="s",
                                  num_cores=SC_CORES,
                                  num_subcores=SC_SUBCORES)
    fn = pl.kernel(
        functools.partial(_sc_s_body, row_base),
        out_type=[jax.ShapeDtypeStruct((N * K, C), jnp.float32),
                  jax.ShapeDtypeStruct((N * K, C), jnp.float32)],
        mesh=mesh,
        scratch_types=[
            pltpu.VMEM((npw * K,), jnp.int32),
            [pltpu.VMEM((K, 2 * C), jnp.float32)] * NSET,
            [pltpu.VMEM((K, C), jnp.float32)] * NSET,
            [pltpu.VMEM((1, 2 * C), jnp.float32)] * NSET,
            [pltpu.VMEM((K, C), jnp.float32)] * NSET,
            [pltpu.SemaphoreType.DMA] * NSET,
            [pltpu.SemaphoreType.DMA] * NSET,
        ],
    )
    return fn(t_tab, pft_tab, qq_tab, idx_flat_b)


def _soft_body(s_ref, ft_ref, out_ref):
    x = s_ref[...]
    m = jnp.max(x, axis=1, keepdims=True)
    e = jnp.exp(x - m)
    den = jnp.sum(e, axis=1)
    num = jnp.sum(e * ft_ref[...], axis=1)
    out_ref[...] = num / den


def _softmax_batch(s_b, ft_b):
    pt = 32
    s3 = s_b.reshape(N, K, C)
    ft3 = ft_b.reshape(N, K, C)
    return pl.pallas_call(
        _soft_body,
        grid=(N // pt,),
        in_specs=[pl.BlockSpec((pt, K, C), lambda r: (r, 0, 0)),
                  pl.BlockSpec((pt, K, C), lambda r: (r, 0, 0))],
        out_specs=pl.BlockSpec((pt, C), lambda r: (r, 0)),
        out_shape=jax.ShapeDtypeStruct((N, C), jnp.float32),
    )(s3, ft3)


def _ln_body(attn_ref, f_ref, g_ref, b_ref, out_ref):
    x = attn_ref[...] + f_ref[...]
    mu = jnp.mean(x, axis=-1, keepdims=True)
    var = jnp.mean((x - mu) ** 2, axis=-1, keepdims=True)
    out_ref[...] = (x - mu) / jnp.sqrt(var + 1e-5) * g_ref[...] + b_ref[...]


def _layernorm(attn, features, gamma, beta):
    total = features.shape[0]
    rb = 512
    row = lambda i: (i, 0)
    rep = lambda i: (0, 0)
    return pl.pallas_call(
        _ln_body,
        grid=(total // rb,),
        in_specs=[
            pl.BlockSpec((rb, C), row),
            pl.BlockSpec((rb, C), row),
            pl.BlockSpec((1, C), rep),
            pl.BlockSpec((1, C), rep),
        ],
        out_specs=pl.BlockSpec((rb, C), row),
        out_shape=jax.ShapeDtypeStruct((total, C), jnp.float32),
    )(attn, features, gamma, beta)


def kernel(features, coords, W_ft, b_ft, W_co, b_co, W_fe, b_fe, gamma, beta):
    total = features.shape[0]
    nb = total // N
    xyz = coords[:, 1:4]
    x2 = jnp.sum(xyz * xyz, axis=1)
    xyzp = jnp.pad(xyz, ((0, 0), (0, 5)))
    W_cop = jnp.pad(W_co, ((0, 5), (0, 0)))

    t_tab, pft_tab, qq_tab = _projections(
        features, xyzp, W_ft, b_ft.reshape(1, C), W_cop, b_co.reshape(1, C),
        W_fe, b_fe.reshape(1, C))

    x2col = x2.reshape(total, 1)
    x2row = x2.reshape(nb, 1, 1, N)
    attns = []
    for b in range(nb):
        idx_b = _knn_batch(xyzp[b * N:(b + 1) * N],
                           x2col[b * N:(b + 1) * N],
                           x2row[b], b * N)
        s_b, ft_b = _sc_s_batch(t_tab, pft_tab, qq_tab,
                                idx_b.reshape(-1), b * N)
        attns.append(_softmax_batch(s_b, ft_b))
    attn = jnp.concatenate(attns, axis=0)

    return _layernorm(attn, features, gamma.reshape(1, C),
                      beta.reshape(1, C))
